# baseline (speedup 1.0000x reference)
import functools

import jax
import jax.numpy as jnp
from jax import lax
from jax.experimental import pallas as pl
from jax.experimental.pallas import tpu as pltpu
from jax.experimental.pallas import tpu_sc as plsc

_D = 32
_V = 8192
_T = 256
_COMMIT = 0.25


def _tc_body(z_ref, cbt_ref, idx_ref, loss_ref, csq_ref):
    i = pl.program_id(0)
    nsteps = pl.num_programs(0)

    @pl.when(i == 0)
    def _init():
        cbt = 0.25 * (cbt_ref[...] * cbt_ref[...])
        csq_ref[...] = jnp.sum(cbt, axis=0, keepdims=True)
        loss_ref[...] = jnp.zeros_like(loss_ref)

    zt = z_ref[...]
    a = jnp.sum(zt * zt, axis=1, keepdims=True)
    m2 = lax.dot_general(zt, cbt_ref[...], (((1,), (0,)), ((), ())),
                         preferred_element_type=jnp.float32)
    d = (a + csq_ref[...]) + m2
    rowmin = jnp.min(d, axis=1, keepdims=True)
    ids = lax.broadcasted_iota(jnp.int32, d.shape, 1)
    cand = jnp.where(d == rowmin, ids, jnp.int32(_V))
    idx_ref[...] = jnp.min(cand, axis=1, keepdims=True)
    loss_ref[...] += jnp.sum(rowmin, axis=0, keepdims=True)

    @pl.when(i == nsteps - 1)
    def _finish():
        n_rows = nsteps * _T
        loss_ref[...] *= (1.0 + _COMMIT) / (n_rows * _D)


def _argmin_and_loss(flat_z, cb_t):
    n = flat_z.shape[0]
    grid = n // _T
    return pl.pallas_call(
        _tc_body,
        grid=(grid,),
        in_specs=[
            pl.BlockSpec((_T, _D), lambda i: (i, 0)),
            pl.BlockSpec((_D, _V), lambda i: (0, 0)),
        ],
        out_specs=[
            pl.BlockSpec((_T, 1), lambda i: (i, 0)),
            pl.BlockSpec((1, 1), lambda i: (0, 0)),
        ],
        out_shape=[
            jax.ShapeDtypeStruct((n, 1), jnp.int32),
            jax.ShapeDtypeStruct((1, 1), jnp.float32),
        ],
        scratch_shapes=[pltpu.VMEM((1, _V), jnp.float32)],
    )(flat_z, cb_t)


_DP = 128


def _sc_gather(codebook_pad, indices):
    n = indices.shape[0]
    info = plsc.get_sparse_core_info()
    nw = info.num_cores * info.num_subcores
    bpw = n // nw
    mesh = plsc.VectorSubcoreMesh(core_axis_name="c", subcore_axis_name="s")

    @functools.partial(
        pl.kernel,
        mesh=mesh,
        out_type=jax.ShapeDtypeStruct((n, _DP), jnp.float32),
        scratch_types=[
            pltpu.VMEM((bpw,), jnp.int32),
            pltpu.VMEM((bpw, _DP), jnp.float32),
            pltpu.SemaphoreType.DMA,
        ],
    )
    def gather(table_hbm, idx_hbm, out_hbm, idx_v, rows_v, sem):
        wid = lax.axis_index("s") * info.num_cores + lax.axis_index("c")
        base = wid * bpw
        pltpu.sync_copy(idx_hbm.at[pl.ds(base, bpw)], idx_v)
        pltpu.async_copy(table_hbm.at[idx_v], rows_v, sem).wait()
        pltpu.sync_copy(rows_v, out_hbm.at[pl.ds(base, bpw)])

    return gather(codebook_pad, indices)


def kernel(z, codebook):
    flat_z = z.reshape(-1, _D)
    cb_t = -2.0 * codebook.T
    idx2, loss2 = _argmin_and_loss(flat_z, cb_t)
    indices = idx2.reshape(-1)
    loss = loss2.reshape(())
    cb_pad = jnp.pad(codebook, ((0, 0), (0, _DP - _D)))
    z_q = _sc_gather(cb_pad, indices)[:, :_D].reshape(z.shape)
    return (z_q, loss, indices)


# speedup vs baseline: 1.1728x; 1.1728x over previous
import functools

import jax
import jax.numpy as jnp
from jax import lax
from jax.experimental import pallas as pl
from jax.experimental.pallas import tpu as pltpu
from jax.experimental.pallas import tpu_sc as plsc

_D = 32
_V = 8192
_T = 256
_COMMIT = 0.25


def _tc_body(z_ref, cbt_ref, idx_ref, loss_ref, csq_ref):
    i = pl.program_id(0)
    nsteps = pl.num_programs(0)

    @pl.when(i == 0)
    def _init():
        cbt = cbt_ref[...]
        csq_ref[...] = jnp.sum(cbt * cbt, axis=0, keepdims=True)
        loss_ref[...] = jnp.zeros_like(loss_ref)

    zt = z_ref[...]
    a = jnp.sum(zt * zt, axis=1, keepdims=True)
    m = lax.dot_general(zt, cbt_ref[...], (((1,), (0,)), ((), ())),
                        preferred_element_type=jnp.float32)
    d = (a + csq_ref[...]) - 2.0 * m
    rowmin = jnp.min(d, axis=1, keepdims=True)
    ids = lax.broadcasted_iota(jnp.int32, d.shape, 1)
    cand = jnp.where(d == rowmin, ids, jnp.int32(_V))
    idx_ref[...] = jnp.min(cand, axis=1, keepdims=True)
    loss_ref[...] += jnp.sum(rowmin, axis=0, keepdims=True)

    @pl.when(i == nsteps - 1)
    def _finish():
        n_rows = nsteps * _T
        loss_ref[...] *= (1.0 + _COMMIT) / (n_rows * _D)


def _argmin_and_loss(flat_z, cb_t):
    n = flat_z.shape[0]
    grid = n // _T
    return pl.pallas_call(
        _tc_body,
        grid=(grid,),
        in_specs=[
            pl.BlockSpec((_T, _D), lambda i: (i, 0)),
            pl.BlockSpec((_D, _V), lambda i: (0, 0)),
        ],
        out_specs=[
            pl.BlockSpec((_T, 1), lambda i: (i, 0)),
            pl.BlockSpec((1, 1), lambda i: (0, 0)),
        ],
        out_shape=[
            jax.ShapeDtypeStruct((n, 1), jnp.int32),
            jax.ShapeDtypeStruct((1, 1), jnp.float32),
        ],
        scratch_shapes=[pltpu.VMEM((1, _V), jnp.float32)],
    )(flat_z, cb_t)


_DP = 128


def _sc_gather(codebook_pad, indices):
    n = indices.shape[0]
    info = plsc.get_sparse_core_info()
    nw = info.num_cores * info.num_subcores
    bpw = n // nw
    mesh = plsc.VectorSubcoreMesh(core_axis_name="c", subcore_axis_name="s")

    @functools.partial(
        pl.kernel,
        mesh=mesh,
        out_type=jax.ShapeDtypeStruct((n, _DP), jnp.float32),
        scratch_types=[
            pltpu.VMEM((bpw,), jnp.int32),
            pltpu.VMEM((bpw, _DP), jnp.float32),
            pltpu.SemaphoreType.DMA,
        ],
    )
    def gather(table_hbm, idx_hbm, out_hbm, idx_v, rows_v, sem):
        wid = lax.axis_index("s") * info.num_cores + lax.axis_index("c")
        base = wid * bpw
        pltpu.sync_copy(idx_hbm.at[pl.ds(base, bpw)], idx_v)
        pltpu.async_copy(table_hbm.at[idx_v], rows_v, sem).wait()
        pltpu.sync_copy(rows_v, out_hbm.at[pl.ds(base, bpw)])

    return gather(codebook_pad, indices)


def kernel(z, codebook):
    flat_z = z.reshape(-1, _D)
    cb_t = codebook.T
    idx2, loss2 = _argmin_and_loss(flat_z, cb_t)
    indices = idx2.reshape(-1)
    loss = loss2.reshape(())
    cb_pad = jnp.pad(codebook, ((0, 0), (0, _DP - _D)))
    z_q = _sc_gather(cb_pad, indices)[:, :_D].reshape(z.shape)
    return (z_q, loss, indices)
